# baseline (speedup 1.0000x reference)
import jax
import jax.numpy as jnp
from jax import lax
from jax.experimental import pallas as pl
from jax.experimental.pallas import tpu as pltpu
from jax.experimental.pallas import tpu_sc as plsc

NUM_EMB = 1000000
DIM = 32
BATCH = 16384
SEQ = 200
B = BATCH * SEQ

NC = 2
NS = 16
NW = NC * NS
BATCH_PER_W = BATCH // NW
CB = 4
CHUNK = CB * SEQ
NBUF = 4
N_CHUNKS = BATCH_PER_W // CB

assert BATCH % NW == 0 and BATCH_PER_W % CB == 0 and N_CHUNKS % NBUF == 0


def _emb_body(idx_hbm, table_hbm, out_hbm, *scratch):
    idx_v = scratch[:NBUF]
    rows_v = scratch[NBUF:2 * NBUF]
    isem = scratch[2 * NBUF:3 * NBUF]
    gsem = scratch[3 * NBUF:4 * NBUF]
    osem = scratch[4 * NBUF:5 * NBUF]

    wid = lax.axis_index("s") * NC + lax.axis_index("c")
    base_b = wid * BATCH_PER_W

    def ichunk(g):
        return idx_hbm.at[pl.ds((base_b + g * CB) * SEQ, CHUNK)]

    def owrite(g, b):
        pltpu.async_copy(
            rows_v[b],
            out_hbm.at[pl.ds((base_b + g * CB) * SEQ, CHUNK), pl.ds(0, DIM)],
            osem[b])

    def owait(g, b):
        pltpu.make_async_copy(
            rows_v[b],
            out_hbm.at[pl.ds((base_b + g * CB) * SEQ, CHUNK), pl.ds(0, DIM)],
            osem[b]).wait()

    for h in range(NBUF):
        pltpu.async_copy(ichunk(h), idx_v[h], isem[h])
    for h in range(NBUF - 1):
        pltpu.make_async_copy(ichunk(h), idx_v[h], isem[h]).wait()
        pltpu.async_copy(table_hbm.at[idx_v[h]], rows_v[h], gsem[h])

    @pl.loop(0, N_CHUNKS, step=NBUF)
    def _(g0):
        for b in range(NBUF):
            g = g0 + b
            pb = (b + NBUF - 1) % NBUF
            pltpu.make_async_copy(
                table_hbm.at[idx_v[b]], rows_v[b], gsem[b]).wait()

            @pl.when(g + NBUF < N_CHUNKS)
            def _():
                pltpu.async_copy(ichunk(g + NBUF), idx_v[b], isem[b])

            @pl.when(g + NBUF - 1 < N_CHUNKS)
            def _():
                pltpu.make_async_copy(
                    ichunk(g + NBUF - 1), idx_v[pb], isem[pb]).wait()

                @pl.when(g >= 1)
                def _():
                    owait(g - 1, pb)

                pltpu.async_copy(
                    table_hbm.at[idx_v[pb]], rows_v[pb], gsem[pb])

            owrite(g, b)

    for h in range(NBUF):
        g = N_CHUNKS - NBUF + h
        owait(g, g % NBUF)


@jax.jit
def _emb_lookup(idx_flat, weight):
    mesh = plsc.VectorSubcoreMesh(core_axis_name="c", subcore_axis_name="s")
    return pl.kernel(
        _emb_body,
        out_type=jax.ShapeDtypeStruct((B, 128), jnp.float32),
        mesh=mesh,
        scratch_types=(
            [pltpu.VMEM((CHUNK,), jnp.int32) for _ in range(NBUF)]
            + [pltpu.VMEM((CHUNK, DIM), jnp.float32) for _ in range(NBUF)]
            + [pltpu.SemaphoreType.DMA for _ in range(3 * NBUF)]
        ),
        compiler_params=pltpu.CompilerParams(use_tc_tiling_on_sc=False),
    )(idx_flat, weight)


K0_R = 320
K0_CHUNKS = NUM_EMB // K0_R


def _depad_body(w_hbm, out_hbm, vin, vout, sem):
    wid = lax.axis_index("s") * NC + lax.axis_index("c")

    @pl.loop(wid, K0_CHUNKS, step=NW)
    def _(g):
        pltpu.sync_copy(
            w_hbm.at[pl.ds(pl.multiple_of(g * K0_R, 8), K0_R)], vin)

        @pl.loop(0, K0_R // 4)
        def _(j):
            for p in range(8):
                vout[j, pl.ds(16 * p, 16)] = (
                    vin[4 * j + p // 2, pl.ds((p % 2) * 16, 16)])

        pltpu.sync_copy(
            vout,
            out_hbm.at[pl.ds(pl.multiple_of(g * (K0_R // 4), 8), K0_R // 4)])


@jax.jit
def _depad(weight):
    mesh = plsc.VectorSubcoreMesh(core_axis_name="c", subcore_axis_name="s")
    return pl.kernel(
        _depad_body,
        out_type=jax.ShapeDtypeStruct((NUM_EMB // 4, 128), jnp.float32),
        mesh=mesh,
        scratch_types=[
            pltpu.VMEM((K0_R, DIM), jnp.float32),
            pltpu.VMEM((K0_R // 4, 128), jnp.float32),
            pltpu.SemaphoreType.DMA,
        ],
        compiler_params=pltpu.CompilerParams(use_tc_tiling_on_sc=True),
    )(weight)


def kernel(token_ids, weight):
    idx_flat = token_ids.reshape(-1).astype(jnp.int32)
    table_lin = _depad(weight).reshape(NUM_EMB, DIM)
    padded = _emb_lookup(idx_flat, table_lin)
    return padded[:, :DIM].reshape(BATCH, SEQ, DIM)


# speedup vs baseline: 1.2065x; 1.2065x over previous
import jax
import jax.numpy as jnp
from jax import lax
from jax.experimental import pallas as pl
from jax.experimental.pallas import tpu as pltpu
from jax.experimental.pallas import tpu_sc as plsc

NUM_EMB = 1000000
DIM = 32
BATCH = 16384
SEQ = 200
B = BATCH * SEQ

NC = 2
NS = 16
NW = NC * NS
BATCH_PER_W = BATCH // NW
CB = 4
CHUNK = CB * SEQ
NBUF = 4
N_CHUNKS = BATCH_PER_W // CB

assert BATCH % NW == 0 and BATCH_PER_W % CB == 0 and N_CHUNKS % NBUF == 0


def _emb_body(idx_hbm, table_hbm, out_hbm, *scratch):
    idx_v = scratch[:NBUF]
    rows_v = scratch[NBUF:2 * NBUF]
    isem = scratch[2 * NBUF:3 * NBUF]
    gsem = scratch[3 * NBUF:4 * NBUF]
    osem = scratch[4 * NBUF:5 * NBUF]

    wid = lax.axis_index("s") * NC + lax.axis_index("c")
    base_b = wid * BATCH_PER_W

    def ichunk(g):
        return idx_hbm.at[pl.ds((base_b + g * CB) * SEQ, CHUNK)]

    def owrite(g, b):
        pltpu.async_copy(
            rows_v[b],
            out_hbm.at[pl.ds((base_b + g * CB) * SEQ, CHUNK), pl.ds(0, DIM)],
            osem[b])

    def owait(g, b):
        pltpu.make_async_copy(
            rows_v[b],
            out_hbm.at[pl.ds((base_b + g * CB) * SEQ, CHUNK), pl.ds(0, DIM)],
            osem[b]).wait()

    for h in range(NBUF):
        pltpu.async_copy(ichunk(h), idx_v[h], isem[h])
    for h in range(NBUF - 1):
        pltpu.make_async_copy(ichunk(h), idx_v[h], isem[h]).wait()
        pltpu.async_copy(table_hbm.at[idx_v[h]], rows_v[h], gsem[h])

    @pl.loop(0, N_CHUNKS, step=NBUF)
    def _(g0):
        for b in range(NBUF):
            g = g0 + b
            pb = (b + NBUF - 1) % NBUF
            pltpu.make_async_copy(
                table_hbm.at[idx_v[b]], rows_v[b], gsem[b]).wait()

            @pl.when(g + NBUF < N_CHUNKS)
            def _():
                pltpu.async_copy(ichunk(g + NBUF), idx_v[b], isem[b])

            @pl.when(g + NBUF - 1 < N_CHUNKS)
            def _():
                pltpu.make_async_copy(
                    ichunk(g + NBUF - 1), idx_v[pb], isem[pb]).wait()

                @pl.when(g >= 1)
                def _():
                    owait(g - 1, pb)

                pltpu.async_copy(
                    table_hbm.at[idx_v[pb]], rows_v[pb], gsem[pb])

            owrite(g, b)

    for h in range(NBUF):
        g = N_CHUNKS - NBUF + h
        owait(g, g % NBUF)


@jax.jit
def _emb_lookup(idx_flat, weight):
    mesh = plsc.VectorSubcoreMesh(core_axis_name="c", subcore_axis_name="s")
    return pl.kernel(
        _emb_body,
        out_type=jax.ShapeDtypeStruct((B, 128), jnp.float32),
        mesh=mesh,
        scratch_types=(
            [pltpu.VMEM((CHUNK,), jnp.int32) for _ in range(NBUF)]
            + [pltpu.VMEM((CHUNK, DIM), jnp.float32) for _ in range(NBUF)]
            + [pltpu.SemaphoreType.DMA for _ in range(3 * NBUF)]
        ),
        compiler_params=pltpu.CompilerParams(use_tc_tiling_on_sc=False),
    )(idx_flat, weight)


def kernel(token_ids, weight):
    idx_flat = token_ids.reshape(-1).astype(jnp.int32)
    w2 = lax.optimization_barrier(weight.reshape(NUM_EMB // 4, 128))
    table_lin = w2.reshape(NUM_EMB, DIM)
    padded = _emb_lookup(idx_flat, table_lin)
    return padded[:, :DIM].reshape(BATCH, SEQ, DIM)
